# speedup vs baseline: 1.7197x; 1.7197x over previous
import jax
import jax.numpy as jnp
from jax.experimental import pallas as pl

BATCH = 4
SEQ_LEN = 8192
D_MODEL = 1024
BLOCK_S = 256


def _add_pe_kernel(x_ref, pe_ref, out_ref):
    out_ref[...] = x_ref[...] + pe_ref[...][None, :, :]


def kernel(x, pos_embed):
    grid = (SEQ_LEN // BLOCK_S,)
    return pl.pallas_call(
        _add_pe_kernel,
        grid=grid,
        in_specs=[
            pl.BlockSpec((BATCH, BLOCK_S, D_MODEL), lambda i: (0, i, 0)),
            pl.BlockSpec((BLOCK_S, D_MODEL), lambda i: (i, 0)),
        ],
        out_specs=pl.BlockSpec((BATCH, BLOCK_S, D_MODEL), lambda i: (0, i, 0)),
        out_shape=jax.ShapeDtypeStruct((BATCH, SEQ_LEN, D_MODEL), x.dtype),
    )(x, pos_embed)


# speedup vs baseline: 1.7309x; 1.0065x over previous
import jax
import jax.numpy as jnp
from jax.experimental import pallas as pl

BATCH = 4
SEQ_LEN = 8192
D_MODEL = 1024
BLOCK_S = 512


def _add_pe_kernel(x_ref, pe_ref, out_ref):
    out_ref[...] = x_ref[...] + pe_ref[...][None, :, :]


def kernel(x, pos_embed):
    grid = (SEQ_LEN // BLOCK_S,)
    return pl.pallas_call(
        _add_pe_kernel,
        grid=grid,
        in_specs=[
            pl.BlockSpec((BATCH, BLOCK_S, D_MODEL), lambda i: (0, i, 0)),
            pl.BlockSpec((BLOCK_S, D_MODEL), lambda i: (i, 0)),
        ],
        out_specs=pl.BlockSpec((BATCH, BLOCK_S, D_MODEL), lambda i: (0, i, 0)),
        out_shape=jax.ShapeDtypeStruct((BATCH, SEQ_LEN, D_MODEL), x.dtype),
    )(x, pos_embed)
